# Initial kernel scaffold; baseline (speedup 1.0000x reference)
import jax
import jax.numpy as jnp
from jax.experimental import pallas as pl


def kernel(test_features, train_features, train_labels):
    raise NotImplementedError("write your pallas kernel here")


# speedup vs baseline: 3.8024x; 3.8024x over previous
import functools

import jax
import jax.numpy as jnp
from jax import lax
from jax.experimental import pallas as pl
from jax.experimental.pallas import tpu as pltpu
from jax.experimental.pallas import tpu_sc as plsc

B = 1024
N = 100000
D = 32
K = 10
C = 50
T_SOFTMAX = 0.07

CHUNK = 128
TILE = 2048
NPAD = 100352
G = NPAD // CHUNK
NTILES = NPAD // TILE
CPG = TILE // CHUNK

NC, NS, L = 2, 16, 16
NW = NC * NS
ROWS_PER_W = B // NW
COUT = 64
NEG = -1e30


def _simk_body(xp_ref, tr_ref, s_ref, cm_ref):
    i = pl.program_id(0)
    s = lax.dot_general(xp_ref[...], tr_ref[...],
                        (((1,), (1,)), ((), ())),
                        preferred_element_type=jnp.float32)

    @pl.when(i < NTILES - 1)
    def _():
        s_ref[...] = s
        cm_ref[0] = jnp.max(s.reshape(B, CPG, CHUNK), axis=2)

    @pl.when(i == NTILES - 1)
    def _():
        cols = i * TILE + lax.broadcasted_iota(jnp.int32, (B, TILE), 1)
        sm = jnp.where(cols >= N, jnp.float32(NEG), s)
        s_ref[...] = sm
        cm_ref[0] = jnp.max(sm.reshape(B, CPG, CHUNK), axis=2)


def _simk(xp, train_p):
    return pl.pallas_call(
        _simk_body,
        grid=(NTILES,),
        out_shape=[
            jax.ShapeDtypeStruct((B, NPAD), jnp.float32),
            jax.ShapeDtypeStruct((NTILES, B, CPG), jnp.float32),
        ],
        in_specs=[
            pl.BlockSpec((B, D), lambda i: (0, 0)),
            pl.BlockSpec((TILE, D), lambda i: (i, 0)),
        ],
        compiler_params=pltpu.CompilerParams(
            dimension_semantics=("arbitrary",)),
        out_specs=[
            pl.BlockSpec((B, TILE), lambda i: (0, i)),
            pl.BlockSpec((1, B, CPG), lambda i: (i, 0, 0)),
        ],
    )(xp, train_p)


def _sort_desc(vals, tags):
    nk, st = plsc.sort_key_val(-vals, tags)
    return -nk, st


def _merge_top16(rv, ri, v, ids):
    sv, si = _sort_desc(v, ids)
    svr = lax.rev(sv, (0,))
    sir = lax.rev(si, (0,))
    take = rv >= svr
    mv = jnp.where(take, rv, svr)
    mi = jnp.where(take, ri, sir)
    return _sort_desc(mv, mi)


def _lane_extract_f(vec, slot):
    return jnp.max(jnp.where(lax.iota(jnp.int32, L) == slot, vec,
                             jnp.float32(-3e38)))


def _lane_extract_i(vec, slot):
    return jnp.max(jnp.where(lax.iota(jnp.int32, L) == slot, vec,
                             jnp.int32(-(2 ** 30))))


def _sck_body(sims_hbm, cmax_hbm, labels_hbm, out_hbm,
              labels_v, cmax_v, cand_v, out_v, sem):
    wid = lax.axis_index("s") * NC + lax.axis_index("c")
    iota = lax.iota(jnp.int32, L)

    pltpu.sync_copy(labels_hbm, labels_v)

    def row_body(rr, _):
        r = wid * ROWS_PER_W + rr
        pltpu.sync_copy(cmax_hbm.at[r], cmax_v)

        def chunk_scan(j, carry):
            rv, ri = carry
            v = cmax_v[pl.ds(j * L, L)]
            ids = j * L + iota
            return _merge_top16(rv, ri, v, ids)

        rv0 = jnp.full((L,), jnp.float32(-3e38))
        ri0 = jnp.zeros((L,), jnp.int32)
        cv, ci = lax.fori_loop(0, G // L, chunk_scan, (rv0, ri0))

        pltpu.async_copy(sims_hbm.at[r * G + ci], cand_v, sem).wait()

        carry = (rv0, ri0)
        for s in range(L):
            cid = _lane_extract_i(ci, s)

            def sub_scan(o, inner, s=s, cid=cid):
                rv, ri = inner
                v = cand_v[s, pl.ds(o * L, L)]
                gids = cid * CHUNK + o * L + iota
                return _merge_top16(rv, ri, v, gids)

            carry = lax.fori_loop(0, CHUNK // L, sub_scan, carry)
        tv, ti = carry

        valid = iota < K
        m = jnp.max(tv)
        e = jnp.where(valid, jnp.exp((tv - m) / jnp.float32(T_SOFTMAX)),
                      jnp.float32(0.0))
        w = e / jnp.sum(e)
        safe = jnp.minimum(jnp.maximum(ti, 0), N - 1)
        lab = plsc.load_gather(labels_v, [safe])

        accs = [jnp.zeros((L,), jnp.float32) for _ in range(COUT // L)]
        for i in range(K):
            wi = _lane_extract_f(w, i)
            li = _lane_extract_i(lab, i)
            for jj in range(COUT // L):
                accs[jj] = accs[jj] + jnp.where(iota + jj * L == li, wi,
                                                jnp.float32(0.0))
        for jj in range(COUT // L):
            out_v[pl.ds(jj * L, L)] = accs[jj]
        pltpu.sync_copy(out_v, out_hbm.at[r])
        return _

    lax.fori_loop(0, ROWS_PER_W, row_body, 0)


def _sck(sims2d, cmax, labels_p):
    mesh = plsc.VectorSubcoreMesh(core_axis_name="c", subcore_axis_name="s",
                                  num_cores=NC, num_subcores=NS)
    f = functools.partial(
        pl.kernel,
        out_type=jax.ShapeDtypeStruct((B, COUT), jnp.float32),
        mesh=mesh,
        scratch_types=[
            pltpu.VMEM((NPAD,), jnp.int32),
            pltpu.VMEM((G,), jnp.float32),
            pltpu.VMEM((L, CHUNK), jnp.float32),
            pltpu.VMEM((COUT,), jnp.float32),
            pltpu.SemaphoreType.DMA,
        ],
        compiler_params=pltpu.CompilerParams(needs_layout_passes=False),
    )(_sck_body)
    return f(sims2d, cmax, labels_p)


def _l2n(x, axis):
    n = jnp.linalg.norm(x, ord=2, axis=axis, keepdims=True)
    return x / jnp.maximum(n, 1e-12)


def kernel(test_features, train_features, train_labels):
    tf = _l2n(train_features.T, axis=1)
    xb = _l2n(test_features, axis=1).astype(jnp.bfloat16)
    tb = jnp.pad(tf.T.astype(jnp.bfloat16), ((0, NPAD - N), (0, 0)))
    labels_p = jnp.pad(train_labels, (0, NPAD - N))
    sims, cmax3 = _simk(xb, tb)
    cmax = jnp.transpose(cmax3, (1, 0, 2)).reshape(B, G)
    sims2d = sims.reshape(B * G, CHUNK)
    out = _sck(sims2d, cmax, labels_p)
    return out[:, :C]


# speedup vs baseline: 3.9552x; 1.0402x over previous
import functools

import jax
import jax.numpy as jnp
from jax import lax
from jax.experimental import pallas as pl
from jax.experimental.pallas import tpu as pltpu
from jax.experimental.pallas import tpu_sc as plsc

B = 1024
N = 100000
D = 32
K = 10
C = 50
T_SOFTMAX = 0.07

CHUNK = 128
TILE = 2048
NPAD = 100352
G = NPAD // CHUNK
NTILES = NPAD // TILE
CPG = TILE // CHUNK

NC, NS, L = 2, 16, 16
NW = NC * NS
ROWS_PER_W = B // NW
COUT = 64
NEG = -1e30


def _simk_body(xp_ref, tr_ref, s_ref, cm_ref, *, b):
    i = pl.program_id(0)
    s = lax.dot_general(xp_ref[...], tr_ref[...],
                        (((1,), (1,)), ((), ())),
                        preferred_element_type=jnp.float32)

    @pl.when(i < NTILES - 1)
    def _():
        s_ref[...] = s
        cm_ref[0] = jnp.max(s.reshape(b, CPG, CHUNK), axis=2)

    @pl.when(i == NTILES - 1)
    def _():
        cols = i * TILE + lax.broadcasted_iota(jnp.int32, (b, TILE), 1)
        sm = jnp.where(cols >= N, jnp.float32(NEG), s)
        s_ref[...] = sm
        cm_ref[0] = jnp.max(sm.reshape(b, CPG, CHUNK), axis=2)


def _simk(xp, train_p):
    b = xp.shape[0]
    return pl.pallas_call(
        functools.partial(_simk_body, b=b),
        grid=(NTILES,),
        out_shape=[
            jax.ShapeDtypeStruct((b, NPAD), jnp.float32),
            jax.ShapeDtypeStruct((NTILES, b, CPG), jnp.float32),
        ],
        in_specs=[
            pl.BlockSpec((b, D), lambda i: (0, 0)),
            pl.BlockSpec((TILE, D), lambda i: (i, 0)),
        ],
        compiler_params=pltpu.CompilerParams(
            dimension_semantics=("arbitrary",)),
        out_specs=[
            pl.BlockSpec((b, TILE), lambda i: (0, i)),
            pl.BlockSpec((1, b, CPG), lambda i: (i, 0, 0)),
        ],
    )(xp, train_p)


def _sort_desc(vals, tags):
    nk, st = plsc.sort_key_val(-vals, tags)
    return -nk, st


def _merge_top16(rv, ri, v, ids):
    sv, si = _sort_desc(v, ids)
    svr = lax.rev(sv, (0,))
    sir = lax.rev(si, (0,))
    take = rv >= svr
    mv = jnp.where(take, rv, svr)
    mi = jnp.where(take, ri, sir)
    return _sort_desc(mv, mi)


def _lane_extract_f(vec, slot):
    return jnp.max(jnp.where(lax.iota(jnp.int32, L) == slot, vec,
                             jnp.float32(-3e38)))


def _lane_extract_i(vec, slot):
    return jnp.max(jnp.where(lax.iota(jnp.int32, L) == slot, vec,
                             jnp.int32(-(2 ** 30))))


def _sck_body(sims_hbm, cmax_hbm, labels_hbm, out_hbm,
              labels_v, cmax_v, cand_v, out_v, sem, *, rows_per_w):
    wid = lax.axis_index("s") * NC + lax.axis_index("c")
    iota = lax.iota(jnp.int32, L)

    pltpu.sync_copy(labels_hbm, labels_v)

    def row_body(rr, _):
        r = wid * rows_per_w + rr
        pltpu.sync_copy(cmax_hbm.at[r], cmax_v)

        def chunk_scan(j, carry):
            rv, ri = carry
            v = cmax_v[pl.ds(j * L, L)]
            ids = j * L + iota
            return _merge_top16(rv, ri, v, ids)

        rv0 = jnp.full((L,), jnp.float32(-3e38))
        ri0 = jnp.zeros((L,), jnp.int32)
        cv, ci = lax.fori_loop(0, G // L, chunk_scan, (rv0, ri0))

        pltpu.async_copy(sims_hbm.at[r * G + ci], cand_v, sem).wait()

        carry = (rv0, ri0)
        for s in range(L):
            cid = _lane_extract_i(ci, s)

            def sub_scan(o, inner, s=s, cid=cid):
                rv, ri = inner
                v = cand_v[s, pl.ds(o * L, L)]
                gids = cid * CHUNK + o * L + iota
                return _merge_top16(rv, ri, v, gids)

            carry = lax.fori_loop(0, CHUNK // L, sub_scan, carry)
        tv, ti = carry

        valid = iota < K
        m = jnp.max(tv)
        e = jnp.where(valid, jnp.exp((tv - m) / jnp.float32(T_SOFTMAX)),
                      jnp.float32(0.0))
        w = e / jnp.sum(e)
        safe = jnp.minimum(jnp.maximum(ti, 0), N - 1)
        lab = plsc.load_gather(labels_v, [safe])

        accs = [jnp.zeros((L,), jnp.float32) for _ in range(COUT // L)]
        for i in range(K):
            wi = _lane_extract_f(w, i)
            li = _lane_extract_i(lab, i)
            for jj in range(COUT // L):
                accs[jj] = accs[jj] + jnp.where(iota + jj * L == li, wi,
                                                jnp.float32(0.0))
        for jj in range(COUT // L):
            out_v[pl.ds(jj * L, L)] = accs[jj]
        pltpu.sync_copy(out_v, out_hbm.at[r])
        return _

    lax.fori_loop(0, rows_per_w, row_body, 0)


def _sck(sims2d, cmax, labels_p):
    b = cmax.shape[0]
    mesh = plsc.VectorSubcoreMesh(core_axis_name="c", subcore_axis_name="s",
                                  num_cores=NC, num_subcores=NS)
    f = functools.partial(
        pl.kernel,
        out_type=jax.ShapeDtypeStruct((b, COUT), jnp.float32),
        mesh=mesh,
        scratch_types=[
            pltpu.VMEM((NPAD,), jnp.int32),
            pltpu.VMEM((G,), jnp.float32),
            pltpu.VMEM((L, CHUNK), jnp.float32),
            pltpu.VMEM((COUT,), jnp.float32),
            pltpu.SemaphoreType.DMA,
        ],
        compiler_params=pltpu.CompilerParams(needs_layout_passes=False),
    )(functools.partial(_sck_body, rows_per_w=b // NW))
    return f(sims2d, cmax, labels_p)


def _l2n(x, axis):
    n = jnp.linalg.norm(x, ord=2, axis=axis, keepdims=True)
    return x / jnp.maximum(n, 1e-12)


def kernel(test_features, train_features, train_labels):
    tf = _l2n(train_features.T, axis=1)
    xb = _l2n(test_features, axis=1).astype(jnp.bfloat16)
    tb = jnp.pad(tf.T.astype(jnp.bfloat16), ((0, NPAD - N), (0, 0)))
    labels_p = jnp.pad(train_labels, (0, NPAD - N))

    outs = []
    h = B // 2
    for i in range(2):
        xh = xb[i * h:(i + 1) * h]
        sims, cmax3 = _simk(xh, tb)
        cmax = jnp.transpose(cmax3, (1, 0, 2)).reshape(h, G)
        sims2d = sims.reshape(h * G, CHUNK)
        outs.append(_sck(sims2d, cmax, labels_p))
    return jnp.concatenate(outs, axis=0)[:, :C]
